# Initial kernel scaffold; baseline (speedup 1.0000x reference)
import functools

import jax
import jax.numpy as jnp
from jax.experimental import pallas as pl

_B = 4096
_NN = 16
_ET = 64
_HID = 128
_HEADS = 4
_NE = 3
_VOCAB = 1000
_FD = 32
_NF = 64
_NC = 16
_BB = 128


def _ln(x, g, b, eps=1e-5):
    m = x.mean(-1, keepdims=True)
    v = ((x - m) ** 2).mean(-1, keepdims=True)
    return (x - m) * jax.lax.rsqrt(v + eps) * g + b


def _dense_gat(x, W, a_s, a_d, bias, heads, od, Crows, mask, bb):
    n = bb * _NN
    xp = jnp.dot(x, W, preferred_element_type=jnp.float32)
    ws_cols = []
    wd_cols = []
    for h in range(heads):
        Wh = W[:, h * od:(h + 1) * od]
        ws_cols.append(jnp.dot(Wh, a_s[h][:, None],
                               preferred_element_type=jnp.float32))
        wd_cols.append(jnp.dot(Wh, a_d[h][:, None],
                               preferred_element_type=jnp.float32))
    Ws = jnp.concatenate(ws_cols, axis=1)
    Wd = jnp.concatenate(wd_cols, axis=1)
    asn = jnp.dot(x, Ws, preferred_element_type=jnp.float32)
    adn = jnp.dot(x, Wd, preferred_element_type=jnp.float32)
    asn3 = asn.reshape(bb, _NN, heads)

    outs = []
    for h in range(heads):
        asn_rep = jnp.broadcast_to(asn3[:, None, :, h],
                                   (bb, _NN, _NN)).reshape(n, _NN)
        L = asn_rep + adn[:, h:h + 1]
        L = jnp.where(L > 0, L, 0.2 * L)
        Lm = jnp.where(mask, L, -1e30)
        m = Lm.max(axis=1, keepdims=True)
        E = Crows * jnp.exp(Lm - m)
        z = E.sum(axis=1, keepdims=True)
        alpha = (E / (z + 1e-16)).reshape(bb, _NN, _NN)
        xph = xp[:, h * od:(h + 1) * od].reshape(bb, _NN, od)
        acc = alpha[:, :, 0:1] * xph[:, 0:1, :]
        for s in range(1, _NN):
            acc = acc + alpha[:, :, s:s + 1] * xph[:, s:s + 1, :]
        outs.append(acc.reshape(n, od))
    out = outs[0] if heads == 1 else jnp.concatenate(outs, axis=1)
    return out + bias[None]


def _body(idx_ref, et_ref, fs_ref, tab_ref, Wa_ref, ba_ref, lng_ref, lnb_ref,
          ml_ref, W1_ref, as1_ref, ad1_ref, b1_ref, W2_ref, as2_ref, ad2_ref,
          b2_ref, gng_ref, gnb_ref, fg_ref, flng_ref, flnb_ref, Wf1_ref,
          bf1_ref, Wf2_ref, bf2_ref, fog_ref, fob_ref, Wg1_ref, bg1_ref,
          Wg2_ref, bg2_ref, out_ref, *, bb):
    n = bb * _NN
    i0 = jax.lax.broadcasted_iota(jnp.int32, (n, _NN), 0)
    i1 = jax.lax.broadcasted_iota(jnp.int32, (n, _NN), 1)
    node_oh = ((i0 & (_NN - 1)) == i1).astype(jnp.float32)

    embs = []
    for e in range(_NE):
        idx = idx_ref[e].reshape(n, 1)
        voc = jax.lax.broadcasted_iota(jnp.int32, (n, _VOCAB), 1)
        onehot = (idx == voc).astype(jnp.float32)
        emb = jnp.dot(onehot, tab_ref[e], preferred_element_type=jnp.float32)
        al = jnp.dot(emb, Wa_ref[e],
                     preferred_element_type=jnp.float32) + ba_ref[e][None]
        alx = _ln(al, lng_ref[e][None], lnb_ref[e][None])
        gate = jax.nn.sigmoid(ml_ref[e])
        gate_col = jnp.dot(node_oh, gate,
                           preferred_element_type=jnp.float32)
        x = alx * gate_col

        src = et_ref[e, 0:1, :]
        dst = et_ref[e, 1:2, :]
        nodes = jax.lax.broadcasted_iota(jnp.int32, (_NN, _ET), 0)
        srcohT = (src == nodes).astype(jnp.float32)
        dstohT = (dst == nodes).astype(jnp.float32)
        eye = (jax.lax.broadcasted_iota(jnp.int32, (_NN, _NN), 0) ==
               jax.lax.broadcasted_iota(jnp.int32, (_NN, _NN), 1)
               ).astype(jnp.float32)
        Ce = jax.lax.dot_general(dstohT, srcohT, (((1,), (1,)), ((), ())),
                                 preferred_element_type=jnp.float32) + eye
        Crows = jnp.dot(node_oh, Ce,
                        preferred_element_type=jnp.float32)
        mask = Crows > 0

        x = _dense_gat(x, W1_ref[e], as1_ref[e], ad1_ref[e], b1_ref[e],
                       _HEADS, _HID, Crows, mask, bb)
        x = jnp.where(x > 0, x, jnp.expm1(jnp.minimum(x, 0.0)))
        x = _dense_gat(x, W2_ref[e], as2_ref[e], ad2_ref[e], b2_ref[e],
                       1, _HID, Crows, mask, bb)
        pooled = x.reshape(bb, _NN, _HID).mean(axis=1)
        embs.append(pooled)

    g = _ln(jnp.concatenate(embs, axis=1), gng_ref[...], gnb_ref[...])
    fgate = jax.nn.sigmoid(fg_ref[...])
    h = _ln(fs_ref[...] * fgate, flng_ref[...], flnb_ref[...])
    h = jnp.dot(h, Wf1_ref[...],
                preferred_element_type=jnp.float32) + bf1_ref[...]
    h = jnp.where(h > 0, h, 0.01 * h)
    h = jnp.dot(h, Wf2_ref[...],
                preferred_element_type=jnp.float32) + bf2_ref[...]
    h = _ln(h, fog_ref[...], fob_ref[...])
    c = jnp.concatenate([g, h], axis=1)
    z = jnp.dot(c, Wg1_ref[...],
                preferred_element_type=jnp.float32) + bg1_ref[...]
    z = jnp.where(z > 0, z, 0.01 * z)
    out_ref[...] = jnp.dot(z, Wg2_ref[...],
                           preferred_element_type=jnp.float32) + bg2_ref[...]


def _full(shape):
    nd = len(shape)
    return pl.BlockSpec(shape, lambda i, _nd=nd: (0,) * _nd)


@jax.jit
def kernel(indices, edge_templates, flow_stats, tables, Wa, ba, lng, lnb,
           mask_logits, W1, as1, ad1, b1, W2, as2, ad2, b2, gng, gnb,
           flow_gate, flng, flnb, Wf1, bf1, Wf2, bf2, fog, fob,
           Wg1, bg1, Wg2, bg2):
    bb = _BB
    nblk = _B // bb
    ml2 = mask_logits.reshape(_NE, _NN, 1)
    args = (indices, edge_templates, flow_stats, tables, Wa, ba, lng, lnb,
            ml2, W1, as1, ad1, b1, W2, as2, ad2, b2,
            gng.reshape(1, -1), gnb.reshape(1, -1), flow_gate.reshape(1, -1),
            flng.reshape(1, -1), flnb.reshape(1, -1), Wf1,
            bf1.reshape(1, -1), Wf2, bf2.reshape(1, -1),
            fog.reshape(1, -1), fob.reshape(1, -1), Wg1,
            bg1.reshape(1, -1), Wg2, bg2.reshape(1, -1))
    in_specs = [
        pl.BlockSpec((_NE, bb, _NN), lambda i: (0, i, 0)),
        _full((_NE, 2, _ET)),
        pl.BlockSpec((bb, _NF), lambda i: (i, 0)),
    ] + [_full(a.shape) for a in args[3:]]
    return pl.pallas_call(
        functools.partial(_body, bb=bb),
        grid=(nblk,),
        in_specs=in_specs,
        out_specs=pl.BlockSpec((bb, _NC), lambda i: (i, 0)),
        out_shape=jax.ShapeDtypeStruct((_B, _NC), jnp.float32),
    )(*args)


# speedup vs baseline: 138.6266x; 138.6266x over previous
import functools

import jax
import jax.numpy as jnp
from jax.experimental import pallas as pl

_B = 4096
_NN = 16
_ET = 64
_HID = 128
_HEADS = 4
_NE = 3
_VOCAB = 1000
_FD = 32
_NF = 64
_NC = 16
_BB = 64


def _ln(x, g, b, eps=1e-5):
    d = x.shape[-1]
    ones_mat = jnp.full((d, d), 1.0 / d, jnp.float32)
    m = jnp.dot(x, ones_mat, preferred_element_type=jnp.float32)
    msq = jnp.dot(x * x, ones_mat, preferred_element_type=jnp.float32)
    v = msq - m * m
    return (x - m) * jax.lax.rsqrt(v + eps) * g + b


_G = 8


def _dense_gat(x, W, a_s, a_d, bias, heads, od, Crows_big, mask_big, bb):
    n = bb * _NN
    ng = n // (_G * _NN)
    xp = jnp.dot(x, W, preferred_element_type=jnp.float32)
    ones_col = jnp.ones((_G * _NN, 1), jnp.float32)

    head_outs = []
    for h in range(heads):
        xph = xp[:, h * od:(h + 1) * od]
        asn_col = (xph * a_s[h][None, :]).sum(axis=1, keepdims=True)
        adn_col = (xph * a_d[h][None, :]).sum(axis=1, keepdims=True)
        asn_big = jnp.concatenate([
            jax.lax.dot_general(ones_col, asn_col[q * 128:(q + 1) * 128, :],
                                (((1,), (1,)), ((), ())),
                                preferred_element_type=jnp.float32)
            for q in range(ng)], axis=0)
        L = asn_big + adn_col
        L = jnp.where(L > 0, L, 0.2 * L)
        Lm = jnp.where(mask_big, L, -1e30)
        m = Lm.max(axis=1, keepdims=True)
        E = Crows_big * jnp.exp(Lm - m)
        z = E.sum(axis=1, keepdims=True)
        alpha = E * (1.0 / (z + 1e-16))
        gouts = [
            jnp.dot(alpha[q * 128:(q + 1) * 128, :],
                    xph[q * 128:(q + 1) * 128, :],
                    preferred_element_type=jnp.float32)
            for q in range(ng)
        ]
        head_outs.append(jnp.concatenate(gouts, axis=0) if ng > 1
                         else gouts[0])
    out = (head_outs[0] if heads == 1
           else jnp.concatenate(head_outs, axis=1))
    return out + bias[None]


def _body(idx_ref, et_ref, fs_ref, tab_ref, Wa_ref, ba_ref, lng_ref, lnb_ref,
          ml_ref, W1_ref, as1_ref, ad1_ref, b1_ref, W2_ref, as2_ref, ad2_ref,
          b2_ref, gng_ref, gnb_ref, fg_ref, flng_ref, flnb_ref, Wf1_ref,
          bf1_ref, Wf2_ref, bf2_ref, fog_ref, fob_ref, Wg1_ref, bg1_ref,
          Wg2_ref, bg2_ref, out_ref, *, bb):
    n = bb * _NN
    i0 = jax.lax.broadcasted_iota(jnp.int32, (n, _NN), 0)
    i1 = jax.lax.broadcasted_iota(jnp.int32, (n, _NN), 1)
    node_oh = ((i0 & (_NN - 1)) == i1).astype(jnp.float32)
    gi0 = jax.lax.broadcasted_iota(jnp.int32, (n, _G * _NN), 0)
    gi1 = jax.lax.broadcasted_iota(jnp.int32, (n, _G * _NN), 1)
    blk_mask = (((gi0 >> 4) & (_G - 1)) == (gi1 >> 4)).astype(jnp.float32)

    embs = []
    for e in range(_NE):
        idxf = idx_ref[e].astype(jnp.float32)
        idx_rep = jnp.broadcast_to(idxf[:, None, :],
                                   (bb, _NN, _NN)).reshape(n, _NN)
        idx_col = (idx_rep * node_oh).sum(axis=1, keepdims=True)
        voc = jax.lax.broadcasted_iota(jnp.int32, (n, _VOCAB), 1)
        onehot = (idx_col.astype(jnp.int32) == voc).astype(jnp.float32)
        emb = jnp.dot(onehot, tab_ref[e], preferred_element_type=jnp.float32)
        al = jnp.dot(emb, Wa_ref[e],
                     preferred_element_type=jnp.float32) + ba_ref[e][None]
        alx = _ln(al, lng_ref[e][None], lnb_ref[e][None])
        gate = jax.nn.sigmoid(ml_ref[e])
        gate_col = jnp.dot(node_oh, gate,
                           preferred_element_type=jnp.float32)
        x = alx * gate_col

        src = et_ref[e, 0:1, :]
        dst = et_ref[e, 1:2, :]
        nodes = jax.lax.broadcasted_iota(jnp.int32, (_NN, _ET), 0)
        srcohT = (src == nodes).astype(jnp.float32)
        dstohT = (dst == nodes).astype(jnp.float32)
        eye = (jax.lax.broadcasted_iota(jnp.int32, (_NN, _NN), 0) ==
               jax.lax.broadcasted_iota(jnp.int32, (_NN, _NN), 1)
               ).astype(jnp.float32)
        Ce = jax.lax.dot_general(dstohT, srcohT, (((1,), (1,)), ((), ())),
                                 preferred_element_type=jnp.float32) + eye
        Ce_big = jnp.concatenate([Ce] * _G, axis=1)
        Crows_big = jnp.dot(node_oh, Ce_big,
                            preferred_element_type=jnp.float32) * blk_mask
        mask_big = Crows_big > 0

        x = _dense_gat(x, W1_ref[e], as1_ref[e], ad1_ref[e], b1_ref[e],
                       _HEADS, _HID, Crows_big, mask_big, bb)
        x = jnp.where(x > 0, x, jnp.exp(jnp.minimum(x, 0.0)) - 1.0)
        x = _dense_gat(x, W2_ref[e], as2_ref[e], ad2_ref[e], b2_ref[e],
                       1, _HID, Crows_big, mask_big, bb)
        pooled = x.reshape(bb, _NN, _HID).mean(axis=1)
        embs.append(pooled)

    g = _ln(jnp.concatenate(embs, axis=1), gng_ref[...], gnb_ref[...])
    fgate = jax.nn.sigmoid(fg_ref[...])
    h = _ln(fs_ref[...] * fgate, flng_ref[...], flnb_ref[...])
    h = jnp.dot(h, Wf1_ref[...],
                preferred_element_type=jnp.float32) + bf1_ref[...]
    h = jnp.where(h > 0, h, 0.01 * h)
    h = jnp.dot(h, Wf2_ref[...],
                preferred_element_type=jnp.float32) + bf2_ref[...]
    h = _ln(h, fog_ref[...], fob_ref[...])
    c = jnp.concatenate([g, h], axis=1)
    z = jnp.dot(c, Wg1_ref[...],
                preferred_element_type=jnp.float32) + bg1_ref[...]
    z = jnp.where(z > 0, z, 0.01 * z)
    out_ref[...] = jnp.dot(z, Wg2_ref[...],
                           preferred_element_type=jnp.float32) + bg2_ref[...]


def _full(shape):
    nd = len(shape)
    return pl.BlockSpec(shape, lambda i, _nd=nd: (0,) * _nd)


@jax.jit
def kernel(indices, edge_templates, flow_stats, tables, Wa, ba, lng, lnb,
           mask_logits, W1, as1, ad1, b1, W2, as2, ad2, b2, gng, gnb,
           flow_gate, flng, flnb, Wf1, bf1, Wf2, bf2, fog, fob,
           Wg1, bg1, Wg2, bg2):
    bb = _BB
    nblk = _B // bb
    ml2 = mask_logits.reshape(_NE, _NN, 1)
    args = (indices, edge_templates, flow_stats, tables, Wa, ba, lng, lnb,
            ml2, W1, as1, ad1, b1, W2, as2, ad2, b2,
            gng.reshape(1, -1), gnb.reshape(1, -1), flow_gate.reshape(1, -1),
            flng.reshape(1, -1), flnb.reshape(1, -1), Wf1,
            bf1.reshape(1, -1), Wf2, bf2.reshape(1, -1),
            fog.reshape(1, -1), fob.reshape(1, -1), Wg1,
            bg1.reshape(1, -1), Wg2, bg2.reshape(1, -1))
    in_specs = [
        pl.BlockSpec((_NE, bb, _NN), lambda i: (0, i, 0)),
        _full((_NE, 2, _ET)),
        pl.BlockSpec((bb, _NF), lambda i: (i, 0)),
    ] + [_full(a.shape) for a in args[3:]]
    return pl.pallas_call(
        functools.partial(_body, bb=bb),
        grid=(nblk,),
        in_specs=in_specs,
        out_specs=pl.BlockSpec((bb, _NC), lambda i: (i, 0)),
        out_shape=jax.ShapeDtypeStruct((_B, _NC), jnp.float32),
    )(*args)


# speedup vs baseline: 160.1718x; 1.1554x over previous
import functools

import jax
import jax.numpy as jnp
from jax.experimental import pallas as pl
from jax.experimental.pallas import tpu as pltpu
from jax.experimental.pallas import tpu_sc as plsc

_B = 4096
_NN = 16
_ET = 64
_HID = 128
_HEADS = 4
_NE = 3
_VOCAB = 1000
_FD = 32
_NF = 64
_NC = 16
_BB = 64


def _ln(x, g, b, eps=1e-5):
    d = x.shape[-1]
    ones_mat = jnp.full((d, d), 1.0 / d, jnp.float32)
    m = jnp.dot(x, ones_mat, preferred_element_type=jnp.float32)
    msq = jnp.dot(x * x, ones_mat, preferred_element_type=jnp.float32)
    v = msq - m * m
    return (x - m) * jax.lax.rsqrt(v + eps) * g + b


_G = 8

_SC_INNER = 6


def _sc_gather(tables_flat, idx_grid, nw):
    total = _NE * _B * _NN
    rpw = total // nw
    outer = rpw // (_SC_INNER * 128)
    info = plsc.get_sparse_core_info()
    nc = info.num_cores
    mesh = plsc.VectorSubcoreMesh(core_axis_name="c", subcore_axis_name="s")

    @functools.partial(
        pl.kernel, mesh=mesh,
        out_type=jax.ShapeDtypeStruct((total, 128), jnp.float32),
        scratch_types=[
            pltpu.VMEM((rpw // 128, 128), jnp.int32),
            pltpu.VMEM((_SC_INNER * 128, 128), jnp.float32),
            pltpu.SemaphoreType.DMA,
        ],
    )
    def k(idx_hbm, tab_hbm, out_hbm, idx_v, rows_v, sem):
        wid = jax.lax.axis_index("s") * nc + jax.lax.axis_index("c")
        pltpu.sync_copy(idx_hbm.at[wid], idx_v)

        def body(j, carry):
            cps = [
                pltpu.async_copy(
                    tab_hbm.at[idx_v.at[j * _SC_INNER + jj]],
                    rows_v.at[pl.ds(jj * 128, 128)], sem)
                for jj in range(_SC_INNER)
            ]
            for cp in cps:
                cp.wait()
            pltpu.sync_copy(
                rows_v,
                out_hbm.at[pl.ds(wid * rpw + j * (_SC_INNER * 128),
                                 _SC_INNER * 128)])
            return carry

        jax.lax.fori_loop(0, outer, body, 0)

    return k(idx_grid, tables_flat)


def _dense_gat(x, W, a_s, a_d, bias, heads, od, Crows_big, mask_big, bb):
    n = bb * _NN
    ng = n // (_G * _NN)
    xp = jnp.dot(x, W, preferred_element_type=jnp.float32)
    ones_col = jnp.ones((_G * _NN, 1), jnp.float32)

    head_outs = []
    for h in range(heads):
        xph = xp[:, h * od:(h + 1) * od]
        asn_col = (xph * a_s[h][None, :]).sum(axis=1, keepdims=True)
        adn_col = (xph * a_d[h][None, :]).sum(axis=1, keepdims=True)
        asn_big = jnp.concatenate([
            jax.lax.dot_general(ones_col, asn_col[q * 128:(q + 1) * 128, :],
                                (((1,), (1,)), ((), ())),
                                preferred_element_type=jnp.float32)
            for q in range(ng)], axis=0)
        L = asn_big + adn_col
        L = jnp.where(L > 0, L, 0.2 * L)
        Lm = jnp.where(mask_big, L, -1e30)
        m = Lm.max(axis=1, keepdims=True)
        E = Crows_big * jnp.exp(Lm - m)
        z = E.sum(axis=1, keepdims=True)
        alpha = E * (1.0 / (z + 1e-16))
        gouts = [
            jnp.dot(alpha[q * 128:(q + 1) * 128, :],
                    xph[q * 128:(q + 1) * 128, :],
                    preferred_element_type=jnp.float32)
            for q in range(ng)
        ]
        head_outs.append(jnp.concatenate(gouts, axis=0) if ng > 1
                         else gouts[0])
    out = (head_outs[0] if heads == 1
           else jnp.concatenate(head_outs, axis=1))
    return out + bias[None]


def _body(emb_ref, et_ref, fs_ref, Wa_ref, ba_ref, lng_ref, lnb_ref,
          ml_ref, W1_ref, as1_ref, ad1_ref, b1_ref, W2_ref, as2_ref, ad2_ref,
          b2_ref, gng_ref, gnb_ref, fg_ref, flng_ref, flnb_ref, Wf1_ref,
          bf1_ref, Wf2_ref, bf2_ref, fog_ref, fob_ref, Wg1_ref, bg1_ref,
          Wg2_ref, bg2_ref, out_ref, *, bb):
    n = bb * _NN
    i0 = jax.lax.broadcasted_iota(jnp.int32, (n, _NN), 0)
    i1 = jax.lax.broadcasted_iota(jnp.int32, (n, _NN), 1)
    node_oh = ((i0 & (_NN - 1)) == i1).astype(jnp.float32)
    gi0 = jax.lax.broadcasted_iota(jnp.int32, (n, _G * _NN), 0)
    gi1 = jax.lax.broadcasted_iota(jnp.int32, (n, _G * _NN), 1)
    blk_mask = (((gi0 >> 4) & (_G - 1)) == (gi1 >> 4)).astype(jnp.float32)

    embs = []
    for e in range(_NE):
        al = jnp.dot(emb_ref[e], Wa_ref[e],
                     preferred_element_type=jnp.float32) + ba_ref[e][None]
        alx = _ln(al, lng_ref[e][None], lnb_ref[e][None])
        gate = jax.nn.sigmoid(ml_ref[e])
        gate_col = jnp.dot(node_oh, gate,
                           preferred_element_type=jnp.float32)
        x = alx * gate_col

        src = et_ref[e, 0:1, :]
        dst = et_ref[e, 1:2, :]
        nodes = jax.lax.broadcasted_iota(jnp.int32, (_NN, _ET), 0)
        srcohT = (src == nodes).astype(jnp.float32)
        dstohT = (dst == nodes).astype(jnp.float32)
        eye = (jax.lax.broadcasted_iota(jnp.int32, (_NN, _NN), 0) ==
               jax.lax.broadcasted_iota(jnp.int32, (_NN, _NN), 1)
               ).astype(jnp.float32)
        Ce = jax.lax.dot_general(dstohT, srcohT, (((1,), (1,)), ((), ())),
                                 preferred_element_type=jnp.float32) + eye
        Ce_big = jnp.concatenate([Ce] * _G, axis=1)
        Crows_big = jnp.dot(node_oh, Ce_big,
                            preferred_element_type=jnp.float32) * blk_mask
        mask_big = Crows_big > 0

        x = _dense_gat(x, W1_ref[e], as1_ref[e], ad1_ref[e], b1_ref[e],
                       _HEADS, _HID, Crows_big, mask_big, bb)
        x = jnp.where(x > 0, x, jnp.exp(jnp.minimum(x, 0.0)) - 1.0)
        x = _dense_gat(x, W2_ref[e], as2_ref[e], ad2_ref[e], b2_ref[e],
                       1, _HID, Crows_big, mask_big, bb)
        pooled = x.reshape(bb, _NN, _HID).mean(axis=1)
        embs.append(pooled)

    g = _ln(jnp.concatenate(embs, axis=1), gng_ref[...], gnb_ref[...])
    fgate = jax.nn.sigmoid(fg_ref[...])
    h = _ln(fs_ref[...] * fgate, flng_ref[...], flnb_ref[...])
    h = jnp.dot(h, Wf1_ref[...],
                preferred_element_type=jnp.float32) + bf1_ref[...]
    h = jnp.where(h > 0, h, 0.01 * h)
    h = jnp.dot(h, Wf2_ref[...],
                preferred_element_type=jnp.float32) + bf2_ref[...]
    h = _ln(h, fog_ref[...], fob_ref[...])
    c = jnp.concatenate([g, h], axis=1)
    z = jnp.dot(c, Wg1_ref[...],
                preferred_element_type=jnp.float32) + bg1_ref[...]
    z = jnp.where(z > 0, z, 0.01 * z)
    out_ref[...] = jnp.dot(z, Wg2_ref[...],
                           preferred_element_type=jnp.float32) + bg2_ref[...]


def _full(shape):
    nd = len(shape)
    return pl.BlockSpec(shape, lambda i, _nd=nd: (0,) * _nd)


@jax.jit
def kernel(indices, edge_templates, flow_stats, tables, Wa, ba, lng, lnb,
           mask_logits, W1, as1, ad1, b1, W2, as2, ad2, b2, gng, gnb,
           flow_gate, flng, flnb, Wf1, bf1, Wf2, bf2, fog, fob,
           Wg1, bg1, Wg2, bg2):
    bb = _BB
    nblk = _B // bb
    nw = 32
    offs = (jnp.arange(_NE, dtype=jnp.int32) * _VOCAB)[:, None]
    idx_grid = (indices.reshape(_NE, _B * _NN) + offs).reshape(nw, -1, 128)
    tab_pad = jnp.concatenate(
        [tables.reshape(_NE * _VOCAB, _FD),
         jnp.zeros((_NE * _VOCAB, 128 - _FD), jnp.float32)], axis=1)
    emb_flat = _sc_gather(tab_pad, idx_grid, nw)
    emb = emb_flat.reshape(_NE, _B * _NN, 128)
    Wa_pad = jnp.concatenate(
        [Wa, jnp.zeros((_NE, 128 - _FD, _HID), jnp.float32)], axis=1)
    ml2 = mask_logits.reshape(_NE, _NN, 1)
    args = (emb, edge_templates, flow_stats, Wa_pad, ba, lng, lnb,
            ml2, W1, as1, ad1, b1, W2, as2, ad2, b2,
            gng.reshape(1, -1), gnb.reshape(1, -1), flow_gate.reshape(1, -1),
            flng.reshape(1, -1), flnb.reshape(1, -1), Wf1,
            bf1.reshape(1, -1), Wf2, bf2.reshape(1, -1),
            fog.reshape(1, -1), fob.reshape(1, -1), Wg1,
            bg1.reshape(1, -1), Wg2, bg2.reshape(1, -1))
    in_specs = [
        pl.BlockSpec((_NE, bb * _NN, 128), lambda i: (0, i, 0)),
        _full((_NE, 2, _ET)),
        pl.BlockSpec((bb, _NF), lambda i: (i, 0)),
    ] + [_full(a.shape) for a in args[3:]]
    return pl.pallas_call(
        functools.partial(_body, bb=bb),
        grid=(nblk,),
        in_specs=in_specs,
        out_specs=pl.BlockSpec((bb, _NC), lambda i: (i, 0)),
        out_shape=jax.ShapeDtypeStruct((_B, _NC), jnp.float32),
    )(*args)


# speedup vs baseline: 205.8994x; 1.2855x over previous
import functools

import jax
import jax.numpy as jnp
from jax.experimental import pallas as pl
from jax.experimental.pallas import tpu as pltpu
from jax.experimental.pallas import tpu_sc as plsc

_B = 4096
_NN = 16
_ET = 64
_HID = 128
_HEADS = 4
_NE = 3
_VOCAB = 1000
_FD = 32
_NF = 64
_NC = 16
_BB = 128


def _ln(x, g, b, eps=1e-5):
    d = x.shape[-1]
    ones_mat = jnp.full((d, d), 1.0 / d, jnp.float32)
    m = jnp.dot(x, ones_mat, preferred_element_type=jnp.float32)
    msq = jnp.dot(x * x, ones_mat, preferred_element_type=jnp.float32)
    v = msq - m * m
    return (x - m) * jax.lax.rsqrt(v + eps) * g + b


_G = 8

_SC_INNER = 6


def _sc_gather(tables_flat, idx_grid, nw):
    total = _NE * _B * _NN
    rpw = total // nw
    outer = rpw // (_SC_INNER * 128)
    info = plsc.get_sparse_core_info()
    nc = info.num_cores
    mesh = plsc.VectorSubcoreMesh(core_axis_name="c", subcore_axis_name="s")

    @functools.partial(
        pl.kernel, mesh=mesh,
        out_type=jax.ShapeDtypeStruct((total, 128), jnp.float32),
        scratch_types=[
            pltpu.VMEM((rpw // 128, 128), jnp.int32),
            pltpu.VMEM((_SC_INNER * 128, 128), jnp.float32),
            pltpu.SemaphoreType.DMA,
        ],
    )
    def k(idx_hbm, tab_hbm, out_hbm, idx_v, rows_v, sem):
        wid = jax.lax.axis_index("s") * nc + jax.lax.axis_index("c")
        pltpu.sync_copy(idx_hbm.at[wid], idx_v)

        def body(j, carry):
            cps = [
                pltpu.async_copy(
                    tab_hbm.at[idx_v.at[j * _SC_INNER + jj]],
                    rows_v.at[pl.ds(jj * 128, 128)], sem)
                for jj in range(_SC_INNER)
            ]
            for cp in cps:
                cp.wait()
            pltpu.sync_copy(
                rows_v,
                out_hbm.at[pl.ds(wid * rpw + j * (_SC_INNER * 128),
                                 _SC_INNER * 128)])
            return carry

        jax.lax.fori_loop(0, outer, body, 0)

    return k(idx_grid, tables_flat)


def _dense_gat(x, W, a_s, a_d, bias, heads, od, Crows_big, bb):
    n = bb * _NN
    ng = n // (_G * _NN)
    xp = jnp.dot(x, W, preferred_element_type=jnp.float32)
    ones_sum = jnp.ones((_G * _NN, _G * _NN), jnp.float32)
    tdims = (((1,), (1,)), ((), ()))

    head_outs = []
    for h in range(heads):
        xph = xp[:, h * od:(h + 1) * od]
        a_s_b = jnp.broadcast_to(a_s[h][None, :], (_G * _NN, od))
        a_d_b = jnp.broadcast_to(a_d[h][None, :], (_G * _NN, od))
        asn_big = jnp.concatenate([
            jax.lax.dot_general(a_s_b, xph[q * 128:(q + 1) * 128, :], tdims,
                                preferred_element_type=jnp.float32)
            for q in range(ng)], axis=0)
        adn_big = jax.lax.dot_general(xph, a_d_b, tdims,
                                      preferred_element_type=jnp.float32)
        L = asn_big + adn_big
        L = jnp.where(L > 0, L, 0.2 * L)
        E = Crows_big * jnp.exp(jnp.minimum(L, 60.0))
        z = jnp.dot(E, ones_sum,
                    preferred_element_type=jnp.float32)
        alpha = E * (1.0 / (z + 1e-16))
        gouts = [
            jnp.dot(alpha[q * 128:(q + 1) * 128, :],
                    xph[q * 128:(q + 1) * 128, :],
                    preferred_element_type=jnp.float32)
            for q in range(ng)
        ]
        head_outs.append(jnp.concatenate(gouts, axis=0) if ng > 1
                         else gouts[0])
    out = (head_outs[0] if heads == 1
           else jnp.concatenate(head_outs, axis=1))
    return out + bias[None]


def _body(emb_ref, et_ref, fs_ref, Wa_ref, ba_ref, lng_ref, lnb_ref,
          ml_ref, W1_ref, as1_ref, ad1_ref, b1_ref, W2_ref, as2_ref, ad2_ref,
          b2_ref, gng_ref, gnb_ref, fg_ref, flng_ref, flnb_ref, Wf1_ref,
          bf1_ref, Wf2_ref, bf2_ref, fog_ref, fob_ref, Wg1_ref, bg1_ref,
          Wg2_ref, bg2_ref, out_ref, *, bb):
    n = bb * _NN
    i0 = jax.lax.broadcasted_iota(jnp.int32, (n, _NN), 0)
    i1 = jax.lax.broadcasted_iota(jnp.int32, (n, _NN), 1)
    node_oh = ((i0 & (_NN - 1)) == i1).astype(jnp.float32)
    gi0 = jax.lax.broadcasted_iota(jnp.int32, (n, _G * _NN), 0)
    gi1 = jax.lax.broadcasted_iota(jnp.int32, (n, _G * _NN), 1)
    blk_mask = (((gi0 >> 4) & (_G - 1)) == (gi1 >> 4)).astype(jnp.float32)

    embs = []
    for e in range(_NE):
        al = jnp.dot(emb_ref[e], Wa_ref[e],
                     preferred_element_type=jnp.float32) + ba_ref[e][None]
        alx = _ln(al, lng_ref[e][None], lnb_ref[e][None])
        gate = jax.nn.sigmoid(ml_ref[e])
        gate_col = jnp.dot(node_oh, gate,
                           preferred_element_type=jnp.float32)
        x = alx * gate_col

        src = et_ref[e, 0:1, :]
        dst = et_ref[e, 1:2, :]
        nodes = jax.lax.broadcasted_iota(jnp.int32, (_NN, _ET), 0)
        srcohT = (src == nodes).astype(jnp.float32)
        dstohT = (dst == nodes).astype(jnp.float32)
        eye = (jax.lax.broadcasted_iota(jnp.int32, (_NN, _NN), 0) ==
               jax.lax.broadcasted_iota(jnp.int32, (_NN, _NN), 1)
               ).astype(jnp.float32)
        Ce = jax.lax.dot_general(dstohT, srcohT, (((1,), (1,)), ((), ())),
                                 preferred_element_type=jnp.float32) + eye
        Ce_big = jnp.concatenate([Ce] * _G, axis=1)
        Crows_big = jnp.dot(node_oh, Ce_big,
                            preferred_element_type=jnp.float32) * blk_mask

        x = _dense_gat(x, W1_ref[e], as1_ref[e], ad1_ref[e], b1_ref[e],
                       _HEADS, _HID, Crows_big, bb)
        x = jnp.where(x > 0, x, jnp.exp(jnp.minimum(x, 0.0)) - 1.0)
        x = _dense_gat(x, W2_ref[e], as2_ref[e], ad2_ref[e], b2_ref[e],
                       1, _HID, Crows_big, bb)
        pooled = x.reshape(bb, _NN, _HID).mean(axis=1)
        embs.append(pooled)

    g = _ln(jnp.concatenate(embs, axis=1), gng_ref[...], gnb_ref[...])
    fgate = jax.nn.sigmoid(fg_ref[...])
    h = _ln(fs_ref[...] * fgate, flng_ref[...], flnb_ref[...])
    h = jnp.dot(h, Wf1_ref[...],
                preferred_element_type=jnp.float32) + bf1_ref[...]
    h = jnp.where(h > 0, h, 0.01 * h)
    h = jnp.dot(h, Wf2_ref[...],
                preferred_element_type=jnp.float32) + bf2_ref[...]
    h = _ln(h, fog_ref[...], fob_ref[...])
    c = jnp.concatenate([g, h], axis=1)
    z = jnp.dot(c, Wg1_ref[...],
                preferred_element_type=jnp.float32) + bg1_ref[...]
    z = jnp.where(z > 0, z, 0.01 * z)
    out_ref[...] = jnp.dot(z, Wg2_ref[...],
                           preferred_element_type=jnp.float32) + bg2_ref[...]


def _full(shape):
    nd = len(shape)
    return pl.BlockSpec(shape, lambda i, _nd=nd: (0,) * _nd)


@jax.jit
def kernel(indices, edge_templates, flow_stats, tables, Wa, ba, lng, lnb,
           mask_logits, W1, as1, ad1, b1, W2, as2, ad2, b2, gng, gnb,
           flow_gate, flng, flnb, Wf1, bf1, Wf2, bf2, fog, fob,
           Wg1, bg1, Wg2, bg2):
    bb = _BB
    nblk = _B // bb
    nw = 32
    offs = (jnp.arange(_NE, dtype=jnp.int32) * _VOCAB)[:, None]
    idx_grid = (indices.reshape(_NE, _B * _NN) + offs).reshape(nw, -1, 128)
    tab_pad = jnp.concatenate(
        [tables.reshape(_NE * _VOCAB, _FD),
         jnp.zeros((_NE * _VOCAB, 128 - _FD), jnp.float32)], axis=1)
    emb_flat = _sc_gather(tab_pad, idx_grid, nw)
    emb = emb_flat.reshape(_NE, _B * _NN, 128)
    Wa_pad = jnp.concatenate(
        [Wa, jnp.zeros((_NE, 128 - _FD, _HID), jnp.float32)], axis=1)
    ml2 = mask_logits.reshape(_NE, _NN, 1)
    args = (emb, edge_templates, flow_stats, Wa_pad, ba, lng, lnb,
            ml2, W1, as1, ad1, b1, W2, as2, ad2, b2,
            gng.reshape(1, -1), gnb.reshape(1, -1), flow_gate.reshape(1, -1),
            flng.reshape(1, -1), flnb.reshape(1, -1), Wf1,
            bf1.reshape(1, -1), Wf2, bf2.reshape(1, -1),
            fog.reshape(1, -1), fob.reshape(1, -1), Wg1,
            bg1.reshape(1, -1), Wg2, bg2.reshape(1, -1))
    in_specs = [
        pl.BlockSpec((_NE, bb * _NN, 128), lambda i: (0, i, 0)),
        _full((_NE, 2, _ET)),
        pl.BlockSpec((bb, _NF), lambda i: (i, 0)),
    ] + [_full(a.shape) for a in args[3:]]
    return pl.pallas_call(
        functools.partial(_body, bb=bb),
        grid=(nblk,),
        in_specs=in_specs,
        out_specs=pl.BlockSpec((bb, _NC), lambda i: (i, 0)),
        out_shape=jax.ShapeDtypeStruct((_B, _NC), jnp.float32),
    )(*args)
